# baseline (speedup 1.0000x reference)
import functools

import jax
import jax.numpy as jnp
from jax import lax
from jax.experimental import pallas as pl
from jax.experimental.pallas import tpu as pltpu
from jax.experimental.pallas import tpu_sc as plsc

_N = 100000
_E = 1600000
_NP = 100096
_SLICE = _NP // 16
_B = 400
_NBI = _SLICE // _B
_TAIL = _SLICE - _NBI * _B
_C = 8
_BD = 2000
_NBD = _SLICE // _BD
_TAILD = _SLICE - _NBD * _BD

_mesh = plsc.VectorSubcoreMesh(core_axis_name="c", subcore_axis_name="s")


def _deg_body(dst_h, ones_h, zeros_h, d0_h, d1_h,
              acc, idx_v, ones_v, buf_v, buf_t):
    c = lax.axis_index("c")
    s = lax.axis_index("s")
    base_r = s * _SLICE

    pltpu.sync_copy(ones_h, ones_v)

    def iblk(k, _):
        pltpu.sync_copy(zeros_h.at[pl.ds(0, _BD)], buf_v)
        pltpu.sync_copy(buf_v, acc.at[pl.ds(base_r + k * _BD, _BD)])
        return _

    lax.fori_loop(0, _NBD, iblk, None)
    pltpu.sync_copy(zeros_h.at[pl.ds(0, _TAILD)], buf_t)
    pltpu.sync_copy(buf_t, acc.at[pl.ds(base_r + _NBD * _BD, _TAILD)])
    plsc.subcore_barrier()

    ebase = (c * 16 + s) * (_E // 32)

    def eblk(k, _):
        pltpu.sync_copy(dst_h.at[pl.ds(ebase + k * _BD, _BD)], idx_v)
        pltpu.sync_copy(ones_v, acc.at[idx_v], add=True)
        return _

    lax.fori_loop(0, (_E // 32) // _BD, eblk, None)
    plsc.subcore_barrier()

    def wout(out_h):
        def wblk(k, _):
            pltpu.sync_copy(acc.at[pl.ds(base_r + k * _BD, _BD)], buf_v)
            pltpu.sync_copy(buf_v, out_h.at[pl.ds(base_r + k * _BD, _BD)])
            return _

        lax.fori_loop(0, _NBD, wblk, None)
        pltpu.sync_copy(acc.at[pl.ds(base_r + _NBD * _BD, _TAILD)], buf_t)
        pltpu.sync_copy(buf_t, out_h.at[pl.ds(base_r + _NBD * _BD, _TAILD)])

    pl.when(c == 0)(lambda: wout(d0_h))
    pl.when(c == 1)(lambda: wout(d1_h))


def _make_deg_kernel():
    return pl.kernel(
        _deg_body,
        out_type=(
            jax.ShapeDtypeStruct((_NP,), jnp.float32),
            jax.ShapeDtypeStruct((_NP,), jnp.float32),
        ),
        mesh=_mesh,
        compiler_params=pltpu.CompilerParams(use_tc_tiling_on_sc=False),
        scratch_types=[
            pltpu.VMEM_SHARED((_NP,), jnp.float32),
            pltpu.VMEM((_BD,), jnp.int32),
            pltpu.VMEM((_BD,), jnp.float32),
            pltpu.VMEM((_BD,), jnp.float32),
            pltpu.VMEM((_TAILD,), jnp.float32),
        ],
    )


def _load_chunk(tab_hbm, tab_s, acc_s, rows_v, rows_t, base_r, also_acc):
    def blk(k, _):
        r0 = base_r + k * _B
        pltpu.sync_copy(tab_hbm.at[pl.ds(r0, _B), :], rows_v)
        pltpu.sync_copy(rows_v, tab_s.at[pl.ds(r0, _B), :])
        if also_acc:
            pltpu.sync_copy(rows_v, acc_s.at[pl.ds(r0, _B), :])
        return _

    lax.fori_loop(0, _NBI, blk, None)
    r0 = base_r + _NBI * _B
    pltpu.sync_copy(tab_hbm.at[pl.ds(r0, _TAIL), :], rows_t)
    pltpu.sync_copy(rows_t, tab_s.at[pl.ds(r0, _TAIL), :])
    if also_acc:
        pltpu.sync_copy(rows_t, acc_s.at[pl.ds(r0, _TAIL), :])


def _zero_acc(zeros_h, acc_s, rows_v, rows_t, base_r):
    def blk(k, _):
        pltpu.sync_copy(zeros_h.at[pl.ds(0, _B), :], rows_v)
        pltpu.sync_copy(rows_v, acc_s.at[pl.ds(base_r + k * _B, _B), :])
        return _

    lax.fori_loop(0, _NBI, blk, None)
    pltpu.sync_copy(zeros_h.at[pl.ds(0, _TAIL), :], rows_t)
    pltpu.sync_copy(rows_t, acc_s.at[pl.ds(base_r + _NBI * _B, _TAIL), :])


def _writeout_acc(acc_s, out_hbm, rows_v, rows_t, base_r):
    def blk(k, _):
        pltpu.sync_copy(acc_s.at[pl.ds(base_r + k * _B, _B), :], rows_v)
        pltpu.sync_copy(rows_v, out_hbm.at[pl.ds(base_r + k * _B, _B), :])
        return _

    lax.fori_loop(0, _NBI, blk, None)
    pltpu.sync_copy(acc_s.at[pl.ds(base_r + _NBI * _B, _TAIL), :], rows_t)
    pltpu.sync_copy(rows_t, out_hbm.at[pl.ds(base_r + _NBI * _B, _TAIL), :])


def _edge_loop(tab_s, src_h, dst_h, acc_s,
               idx_s0, idx_s1, idx_d, rows0, rows1, sem0, sem1,
               ebase, nblk):
    npair = nblk // 2
    leftover = nblk % 2

    def pair(j, _):
        e0 = ebase + (2 * j) * _B
        e1 = e0 + _B
        pltpu.sync_copy(src_h.at[pl.ds(e0, _B)], idx_s0)
        d0 = pltpu.async_copy(tab_s.at[idx_s0], rows0, sem0)
        pltpu.sync_copy(src_h.at[pl.ds(e1, _B)], idx_s1)
        d1 = pltpu.async_copy(tab_s.at[idx_s1], rows1, sem1)
        d0.wait()
        pltpu.sync_copy(dst_h.at[pl.ds(e0, _B)], idx_d)
        pltpu.sync_copy(rows0, acc_s.at[idx_d], add=True)
        d1.wait()
        pltpu.sync_copy(dst_h.at[pl.ds(e1, _B)], idx_d)
        pltpu.sync_copy(rows1, acc_s.at[idx_d], add=True)
        return _

    lax.fori_loop(0, npair, pair, None)

    if leftover:
        e0 = ebase + (nblk - 1) * _B
        pltpu.sync_copy(src_h.at[pl.ds(e0, _B)], idx_s0)
        pltpu.async_copy(tab_s.at[idx_s0], rows0, sem0).wait()
        pltpu.sync_copy(dst_h.at[pl.ds(e0, _B)], idx_d)
        pltpu.sync_copy(rows0, acc_s.at[idx_d], add=True)


def _prop1_body(t0, t1, t2, t3, t4, t5, t6, t7, src_h, dst_h,
                o0, o1, o2, o3, o4, o5, o6, o7,
                tab_s, acc_s, idx_s0, idx_s1, idx_d, rows0, rows1, rows_t,
                sem0, sem1):
    c = lax.axis_index("c")
    s = lax.axis_index("s")
    base_r = s * _SLICE
    nblk = _E // 16 // _B
    ebase = s * (_E // 16)

    def run(tab_h, out_h):
        _load_chunk(tab_h, tab_s, acc_s, rows0, rows_t, base_r, True)
        plsc.subcore_barrier()
        _edge_loop(tab_s, src_h, dst_h, acc_s,
                   idx_s0, idx_s1, idx_d, rows0, rows1, sem0, sem1,
                   ebase, nblk)
        plsc.subcore_barrier()
        _writeout_acc(acc_s, out_h, rows0, rows_t, base_r)
        plsc.subcore_barrier()

    def core0():
        run(t0, o0)
        run(t1, o1)
        run(t2, o2)
        run(t3, o3)

    def core1():
        run(t4, o4)
        run(t5, o5)
        run(t6, o6)
        run(t7, o7)

    pl.when(c == 0)(core0)
    pl.when(c == 1)(core1)


def _make_prop1():
    return pl.kernel(
        _prop1_body,
        out_type=tuple(
            jax.ShapeDtypeStruct((_NP, _C), jnp.float32) for _ in range(8)
        ),
        mesh=_mesh,
        compiler_params=pltpu.CompilerParams(use_tc_tiling_on_sc=False),
        scratch_types=[
            pltpu.VMEM_SHARED((_NP, _C), jnp.float32),
            pltpu.VMEM_SHARED((_NP, _C), jnp.float32),
            pltpu.VMEM((_B,), jnp.int32),
            pltpu.VMEM((_B,), jnp.int32),
            pltpu.VMEM((_B,), jnp.int32),
            pltpu.VMEM((_B, _C), jnp.float32),
            pltpu.VMEM((_B, _C), jnp.float32),
            pltpu.VMEM((_TAIL, _C), jnp.float32),
            pltpu.SemaphoreType.DMA,
            pltpu.SemaphoreType.DMA,
        ],
    )


def _prop2_body(tab_h, zeros_h, src_h, dst_h, p0_h, p1_h,
                tab_s, acc_s, idx_s0, idx_s1, idx_d, rows0, rows1, rows_t,
                sem0, sem1):
    c = lax.axis_index("c")
    s = lax.axis_index("s")
    base_r = s * _SLICE
    epc = _E // 2
    nblk = epc // 16 // _B

    def run(zero_init, out_h, ebase0):
        _load_chunk(tab_h, tab_s, acc_s, rows0, rows_t, base_r,
                    not zero_init)
        if zero_init:
            _zero_acc(zeros_h, acc_s, rows0, rows_t, base_r)
        plsc.subcore_barrier()
        _edge_loop(tab_s, src_h, dst_h, acc_s,
                   idx_s0, idx_s1, idx_d, rows0, rows1, sem0, sem1,
                   ebase0 + s * (epc // 16), nblk)
        plsc.subcore_barrier()
        _writeout_acc(acc_s, out_h, rows0, rows_t, base_r)

    pl.when(c == 0)(lambda: run(False, p0_h, 0))
    pl.when(c == 1)(lambda: run(True, p1_h, epc))


def _make_prop2():
    return pl.kernel(
        _prop2_body,
        out_type=(
            jax.ShapeDtypeStruct((_NP, _C), jnp.float32),
            jax.ShapeDtypeStruct((_NP, _C), jnp.float32),
        ),
        mesh=_mesh,
        compiler_params=pltpu.CompilerParams(use_tc_tiling_on_sc=False),
        scratch_types=[
            pltpu.VMEM_SHARED((_NP, _C), jnp.float32),
            pltpu.VMEM_SHARED((_NP, _C), jnp.float32),
            pltpu.VMEM((_B,), jnp.int32),
            pltpu.VMEM((_B,), jnp.int32),
            pltpu.VMEM((_B,), jnp.int32),
            pltpu.VMEM((_B, _C), jnp.float32),
            pltpu.VMEM((_B, _C), jnp.float32),
            pltpu.VMEM((_TAIL, _C), jnp.float32),
            pltpu.SemaphoreType.DMA,
            pltpu.SemaphoreType.DMA,
        ],
    )


def _dinv_kernel(d0_ref, d1_ref, o_ref):
    o_ref[...] = lax.rsqrt(d0_ref[...] + d1_ref[...] + 1.0)


def _mm_kernel(x_ref, w_ref, o_ref):
    o_ref[...] = jnp.dot(x_ref[...], w_ref[...],
                         preferred_element_type=jnp.float32)


def _scale_split_kernel(h_ref, dinv_ref, *outs):
    g = h_ref[...] * dinv_ref[...]
    for i, o in enumerate(outs):
        o[...] = g[:, i * _C:(i + 1) * _C]


def _mid_kernel(a0, a1, a2, a3, a4, a5, a6, a7, dinv_ref, b1_ref, w2_ref,
                o_ref):
    agg = jnp.concatenate(
        [a0[...], a1[...], a2[...], a3[...],
         a4[...], a5[...], a6[...], a7[...]], axis=1)
    h1 = jax.nn.relu(agg * dinv_ref[...] + b1_ref[...])
    g2 = jnp.dot(h1, w2_ref[...], preferred_element_type=jnp.float32)
    o_ref[...] = g2 * dinv_ref[...]


def _final_kernel(p0, p1, dinv_ref, b2_ref, o_ref):
    v = (p0[...] + p1[...]) * dinv_ref[...]
    o_ref[...] = jax.nn.sigmoid(v[:, 0:2] + b2_ref[...])


def kernel(x, edge_index, W1, b1, W2, b2):
    src = edge_index[0]
    dst = edge_index[1]

    ones1 = jnp.ones((_BD,), jnp.float32)
    zeros1 = jnp.zeros((_BD,), jnp.float32)
    zeros2 = jnp.zeros((_B, _C), jnp.float32)

    grid = _N // 2000
    tab_sds = jax.ShapeDtypeStruct((_NP, _C), jnp.float32)
    tab_spec = pl.BlockSpec((2000, _C), lambda i: (i, 0))
    dinv_spec = pl.BlockSpec((2000, 1), lambda i: (i, 0))

    W1p = jnp.pad(W1, ((0, 0), (0, 14)))
    h = pl.pallas_call(
        _mm_kernel,
        grid=(grid,),
        in_specs=[
            pl.BlockSpec((2000, 768), lambda i: (i, 0)),
            pl.BlockSpec((768, 64), lambda i: (0, 0)),
        ],
        out_specs=pl.BlockSpec((2000, 64), lambda i: (i, 0)),
        out_shape=jax.ShapeDtypeStruct((_N, 64), jnp.float32),
    )(x, W1p)

    d0, d1 = _make_deg_kernel()(dst, ones1, zeros1)

    dinvp = pl.pallas_call(
        _dinv_kernel,
        out_shape=jax.ShapeDtypeStruct((782, 128), jnp.float32),
    )(d0.reshape(782, 128), d1.reshape(782, 128))
    dinv2d = dinvp.reshape(_NP, 1)

    g1c = pl.pallas_call(
        _scale_split_kernel,
        grid=(grid,),
        in_specs=[
            pl.BlockSpec((2000, 64), lambda i: (i, 0)),
            dinv_spec,
        ],
        out_specs=[tab_spec] * 8,
        out_shape=[tab_sds] * 8,
    )(h, dinv2d)

    agg = _make_prop1()(*g1c, src, dst)

    b1p = jnp.pad(b1, (0, 14)).reshape(1, 64)
    W2p = jnp.pad(W2, ((0, 14), (0, 6)))
    g2 = pl.pallas_call(
        _mid_kernel,
        grid=(grid,),
        in_specs=[tab_spec] * 8 + [
            dinv_spec,
            pl.BlockSpec((1, 64), lambda i: (0, 0)),
            pl.BlockSpec((64, _C), lambda i: (0, 0))],
        out_specs=tab_spec,
        out_shape=tab_sds,
    )(*agg, dinv2d, b1p, W2p)

    p0, p1 = _make_prop2()(g2, zeros2, src, dst)

    b2r = b2.reshape(1, 2)
    out = pl.pallas_call(
        _final_kernel,
        grid=(grid,),
        in_specs=[tab_spec, tab_spec, dinv_spec,
                  pl.BlockSpec((1, 2), lambda i: (0, 0))],
        out_specs=pl.BlockSpec((2000, 2), lambda i: (i, 0)),
        out_shape=jax.ShapeDtypeStruct((_N, 2), jnp.float32),
    )(p0, p1, dinv2d, b2r)
    return out


# speedup vs baseline: 1.3521x; 1.3521x over previous
import functools

import jax
import jax.numpy as jnp
from jax import lax
from jax.experimental import pallas as pl
from jax.experimental.pallas import tpu as pltpu
from jax.experimental.pallas import tpu_sc as plsc

_N = 100000
_E = 1600000
_NP = 100096
_SLICE = _NP // 16
_B = 400
_NBI = _SLICE // _B
_TAIL = _SLICE - _NBI * _B
_BD = 2000
_NBD = _SLICE // _BD
_TAILD = _SLICE - _NBD * _BD

_mesh = plsc.VectorSubcoreMesh(core_axis_name="c", subcore_axis_name="s")


def _deg_body(dst_h, ones_h, zeros_h, d0_h, d1_h,
              acc, idx_v, ones_v, buf_v, buf_t):
    c = lax.axis_index("c")
    s = lax.axis_index("s")
    base_r = s * _SLICE

    pltpu.sync_copy(ones_h, ones_v)

    def iblk(k, _):
        pltpu.sync_copy(zeros_h.at[pl.ds(0, _BD)], buf_v)
        pltpu.sync_copy(buf_v, acc.at[pl.ds(base_r + k * _BD, _BD)])
        return _

    lax.fori_loop(0, _NBD, iblk, None)
    pltpu.sync_copy(zeros_h.at[pl.ds(0, _TAILD)], buf_t)
    pltpu.sync_copy(buf_t, acc.at[pl.ds(base_r + _NBD * _BD, _TAILD)])
    plsc.subcore_barrier()

    ebase = (c * 16 + s) * (_E // 32)

    def eblk(k, _):
        pltpu.sync_copy(dst_h.at[pl.ds(ebase + k * _BD, _BD)], idx_v)
        pltpu.sync_copy(ones_v, acc.at[idx_v], add=True)
        return _

    lax.fori_loop(0, (_E // 32) // _BD, eblk, None)
    plsc.subcore_barrier()

    def wout(out_h):
        def wblk(k, _):
            pltpu.sync_copy(acc.at[pl.ds(base_r + k * _BD, _BD)], buf_v)
            pltpu.sync_copy(buf_v, out_h.at[pl.ds(base_r + k * _BD, _BD)])
            return _

        lax.fori_loop(0, _NBD, wblk, None)
        pltpu.sync_copy(acc.at[pl.ds(base_r + _NBD * _BD, _TAILD)], buf_t)
        pltpu.sync_copy(buf_t, out_h.at[pl.ds(base_r + _NBD * _BD, _TAILD)])

    pl.when(c == 0)(lambda: wout(d0_h))
    pl.when(c == 1)(lambda: wout(d1_h))


def _make_deg_kernel():
    return pl.kernel(
        _deg_body,
        out_type=(
            jax.ShapeDtypeStruct((_NP,), jnp.float32),
            jax.ShapeDtypeStruct((_NP,), jnp.float32),
        ),
        mesh=_mesh,
        compiler_params=pltpu.CompilerParams(use_tc_tiling_on_sc=False),
        scratch_types=[
            pltpu.VMEM_SHARED((_NP,), jnp.float32),
            pltpu.VMEM((_BD,), jnp.int32),
            pltpu.VMEM((_BD,), jnp.float32),
            pltpu.VMEM((_BD,), jnp.float32),
            pltpu.VMEM((_TAILD,), jnp.float32),
        ],
    )


def _init_acc2d(src_hbm, acc, rows_v, rows_t, base_r, fixed_block):
    def blk(k, _):
        r0 = 0 if fixed_block else base_r + k * _B
        pltpu.sync_copy(src_hbm.at[pl.ds(r0, _B), :], rows_v)
        pltpu.sync_copy(rows_v, acc.at[pl.ds(base_r + k * _B, _B), :])
        return _

    lax.fori_loop(0, _NBI, blk, None)
    r0 = 0 if fixed_block else base_r + _NBI * _B
    pltpu.sync_copy(src_hbm.at[pl.ds(r0, _TAIL), :], rows_t)
    pltpu.sync_copy(rows_t, acc.at[pl.ds(base_r + _NBI * _B, _TAIL), :])


def _writeout_acc2d(acc, out_hbm, rows_v, rows_t, base_r):
    def blk(k, _):
        pltpu.sync_copy(acc.at[pl.ds(base_r + k * _B, _B), :], rows_v)
        pltpu.sync_copy(rows_v, out_hbm.at[pl.ds(base_r + k * _B, _B), :])
        return _

    lax.fori_loop(0, _NBI, blk, None)
    pltpu.sync_copy(acc.at[pl.ds(base_r + _NBI * _B, _TAIL), :], rows_t)
    pltpu.sync_copy(rows_t, out_hbm.at[pl.ds(base_r + _NBI * _B, _TAIL), :])


def _edge_loop(tab, src_h, dst_h, acc,
               idx_s0, idx_s1, idx_d, rows0, rows1, sem0, sem1,
               ebase, nblk):
    npair = nblk // 2
    leftover = nblk % 2

    def pair(j, _):
        e0 = ebase + (2 * j) * _B
        e1 = e0 + _B
        pltpu.sync_copy(src_h.at[pl.ds(e0, _B)], idx_s0)
        d0 = pltpu.async_copy(tab.at[idx_s0], rows0, sem0)
        pltpu.sync_copy(src_h.at[pl.ds(e1, _B)], idx_s1)
        d1 = pltpu.async_copy(tab.at[idx_s1], rows1, sem1)
        d0.wait()
        pltpu.sync_copy(dst_h.at[pl.ds(e0, _B)], idx_d)
        pltpu.sync_copy(rows0, acc.at[idx_d], add=True)
        d1.wait()
        pltpu.sync_copy(dst_h.at[pl.ds(e1, _B)], idx_d)
        pltpu.sync_copy(rows1, acc.at[idx_d], add=True)
        return _

    lax.fori_loop(0, npair, pair, None)

    if leftover:
        e0 = ebase + (nblk - 1) * _B
        pltpu.sync_copy(src_h.at[pl.ds(e0, _B)], idx_s0)
        pltpu.async_copy(tab.at[idx_s0], rows0, sem0).wait()
        pltpu.sync_copy(dst_h.at[pl.ds(e0, _B)], idx_d)
        pltpu.sync_copy(rows0, acc.at[idx_d], add=True)


def _prop1_body(t0_h, t1_h, t2_h, t3_h, src_h, dst_h,
                o0_h, o1_h, o2_h, o3_h,
                acc, idx_s0, idx_s1, idx_d, rows0, rows1, rows_t,
                sem0, sem1):
    c = lax.axis_index("c")
    s = lax.axis_index("s")
    base_r = s * _SLICE
    nblk = _E // 16 // _B
    ebase = s * (_E // 16)

    def run(tab, out_h):
        _init_acc2d(tab, acc, rows0, rows_t, base_r, False)
        plsc.subcore_barrier()
        _edge_loop(tab, src_h, dst_h, acc,
                   idx_s0, idx_s1, idx_d, rows0, rows1, sem0, sem1,
                   ebase, nblk)
        plsc.subcore_barrier()
        _writeout_acc2d(acc, out_h, rows0, rows_t, base_r)
        plsc.subcore_barrier()

    def core0():
        run(t0_h, o0_h)
        run(t1_h, o1_h)

    def core1():
        run(t2_h, o2_h)
        run(t3_h, o3_h)

    pl.when(c == 0)(core0)
    pl.when(c == 1)(core1)


def _make_prop1():
    return pl.kernel(
        _prop1_body,
        out_type=tuple(
            jax.ShapeDtypeStruct((_NP, 16), jnp.float32) for _ in range(4)
        ),
        mesh=_mesh,
        compiler_params=pltpu.CompilerParams(use_tc_tiling_on_sc=False),
        scratch_types=[
            pltpu.VMEM_SHARED((_NP, 16), jnp.float32),
            pltpu.VMEM((_B,), jnp.int32),
            pltpu.VMEM((_B,), jnp.int32),
            pltpu.VMEM((_B,), jnp.int32),
            pltpu.VMEM((_B, 16), jnp.float32),
            pltpu.VMEM((_B, 16), jnp.float32),
            pltpu.VMEM((_TAIL, 16), jnp.float32),
            pltpu.SemaphoreType.DMA,
            pltpu.SemaphoreType.DMA,
        ],
    )


def _prop2_body(tab_h, zeros_h, src_h, dst_h, p0_h, p1_h,
                acc, idx_s0, idx_s1, idx_d, rows0, rows1, rows_t,
                sem0, sem1):
    c = lax.axis_index("c")
    s = lax.axis_index("s")
    base_r = s * _SLICE
    epc = _E // 2
    nblk = epc // 16 // _B

    def run(init_h, out_h, fixed, ebase0):
        _init_acc2d(init_h, acc, rows0, rows_t, base_r, fixed)
        plsc.subcore_barrier()
        _edge_loop(tab_h, src_h, dst_h, acc,
                   idx_s0, idx_s1, idx_d, rows0, rows1, sem0, sem1,
                   ebase0 + s * (epc // 16), nblk)
        plsc.subcore_barrier()
        _writeout_acc2d(acc, out_h, rows0, rows_t, base_r)

    pl.when(c == 0)(lambda: run(tab_h, p0_h, False, 0))
    pl.when(c == 1)(lambda: run(zeros_h, p1_h, True, epc))


def _make_prop2():
    return pl.kernel(
        _prop2_body,
        out_type=(
            jax.ShapeDtypeStruct((_NP, 16), jnp.float32),
            jax.ShapeDtypeStruct((_NP, 16), jnp.float32),
        ),
        mesh=_mesh,
        compiler_params=pltpu.CompilerParams(use_tc_tiling_on_sc=False),
        scratch_types=[
            pltpu.VMEM_SHARED((_NP, 16), jnp.float32),
            pltpu.VMEM((_B,), jnp.int32),
            pltpu.VMEM((_B,), jnp.int32),
            pltpu.VMEM((_B,), jnp.int32),
            pltpu.VMEM((_B, 16), jnp.float32),
            pltpu.VMEM((_B, 16), jnp.float32),
            pltpu.VMEM((_TAIL, 16), jnp.float32),
            pltpu.SemaphoreType.DMA,
            pltpu.SemaphoreType.DMA,
        ],
    )


def _mm_scale_kernel(x_ref, w_ref, d0_ref, d1_ref, o0, o1, o2, o3):
    h = jnp.dot(x_ref[...], w_ref[...], preferred_element_type=jnp.float32)
    dinv = lax.rsqrt(d0_ref[...] + d1_ref[...] + 1.0)
    g = h * dinv
    o0[...] = g[:, 0:16]
    o1[...] = g[:, 16:32]
    o2[...] = g[:, 32:48]
    o3[...] = g[:, 48:64]


def _mid_kernel(a0, a1, a2, a3, d0_ref, d1_ref, b1_ref, w2_ref, o_ref):
    agg = jnp.concatenate([a0[...], a1[...], a2[...], a3[...]], axis=1)
    dinv = lax.rsqrt(d0_ref[...] + d1_ref[...] + 1.0)
    h1 = jax.nn.relu(agg * dinv + b1_ref[...])
    g2 = jnp.dot(h1, w2_ref[...], preferred_element_type=jnp.float32)
    o_ref[...] = g2 * dinv


def _final_kernel(p0, p1, d0_ref, d1_ref, b2_ref, o_ref):
    dinv = lax.rsqrt(d0_ref[...] + d1_ref[...] + 1.0)
    v = (p0[...] + p1[...]) * dinv
    o_ref[...] = jax.nn.sigmoid(v[:, 0:2] + b2_ref[...])


def kernel(x, edge_index, W1, b1, W2, b2):
    src = edge_index[0]
    dst = edge_index[1]

    ones1 = jnp.ones((_BD,), jnp.float32)
    zeros1 = jnp.zeros((_BD,), jnp.float32)
    zeros2 = jnp.zeros((_B, 16), jnp.float32)

    grid = _N // 2000
    tab_sds = jax.ShapeDtypeStruct((_NP, 16), jnp.float32)
    tab_spec = pl.BlockSpec((2000, 16), lambda i: (i, 0))
    dinv_spec = pl.BlockSpec((2000, 1), lambda i: (i, 0))

    d0, d1 = _make_deg_kernel()(dst, ones1, zeros1)
    d0r = d0.reshape(_NP, 1)
    d1r = d1.reshape(_NP, 1)

    W1p = jnp.pad(W1, ((0, 0), (0, 14)))
    g1c = pl.pallas_call(
        _mm_scale_kernel,
        grid=(grid,),
        in_specs=[
            pl.BlockSpec((2000, 768), lambda i: (i, 0)),
            pl.BlockSpec((768, 64), lambda i: (0, 0)),
            dinv_spec,
            dinv_spec,
        ],
        out_specs=[tab_spec] * 4,
        out_shape=[tab_sds] * 4,
    )(x, W1p, d0r, d1r)

    agg = _make_prop1()(g1c[0], g1c[1], g1c[2], g1c[3], src, dst)

    b1p = jnp.pad(b1, (0, 14)).reshape(1, 64)
    W2p = jnp.pad(W2, ((0, 14), (0, 14)))
    g2 = pl.pallas_call(
        _mid_kernel,
        grid=(grid,),
        in_specs=[tab_spec, tab_spec, tab_spec, tab_spec,
                  dinv_spec, dinv_spec,
                  pl.BlockSpec((1, 64), lambda i: (0, 0)),
                  pl.BlockSpec((64, 16), lambda i: (0, 0))],
        out_specs=tab_spec,
        out_shape=tab_sds,
    )(agg[0], agg[1], agg[2], agg[3], d0r, d1r, b1p, W2p)

    p0, p1 = _make_prop2()(g2, zeros2, src, dst)

    b2r = b2.reshape(1, 2)
    out = pl.pallas_call(
        _final_kernel,
        grid=(grid,),
        in_specs=[tab_spec, tab_spec, dinv_spec, dinv_spec,
                  pl.BlockSpec((1, 2), lambda i: (0, 0))],
        out_specs=pl.BlockSpec((2000, 2), lambda i: (i, 0)),
        out_shape=jax.ShapeDtypeStruct((_N, 2), jnp.float32),
    )(p0, p1, d0r, d1r, b2r)
    return out


# speedup vs baseline: 1.4582x; 1.0785x over previous
import functools

import jax
import jax.numpy as jnp
from jax import lax
from jax.experimental import pallas as pl
from jax.experimental.pallas import tpu as pltpu
from jax.experimental.pallas import tpu_sc as plsc

_N = 100000
_E = 1600000
_NP = 100096
_SLICE = _NP // 16
_B = 400
_NBI = _SLICE // _B
_TAIL = _SLICE - _NBI * _B
_BD = 2000
_NBD = _SLICE // _BD
_TAILD = _SLICE - _NBD * _BD

_mesh = plsc.VectorSubcoreMesh(core_axis_name="c", subcore_axis_name="s")


def _deg_body(dst_h, ones_h, zeros_h, d0_h, d1_h,
              acc, idx_v, ones_v, buf_v, buf_t):
    c = lax.axis_index("c")
    s = lax.axis_index("s")
    base_r = s * _SLICE

    pltpu.sync_copy(ones_h, ones_v)

    def iblk(k, _):
        pltpu.sync_copy(zeros_h.at[pl.ds(0, _BD)], buf_v)
        pltpu.sync_copy(buf_v, acc.at[pl.ds(base_r + k * _BD, _BD)])
        return _

    lax.fori_loop(0, _NBD, iblk, None)
    pltpu.sync_copy(zeros_h.at[pl.ds(0, _TAILD)], buf_t)
    pltpu.sync_copy(buf_t, acc.at[pl.ds(base_r + _NBD * _BD, _TAILD)])
    plsc.subcore_barrier()

    ebase = (c * 16 + s) * (_E // 32)

    def eblk(k, _):
        pltpu.sync_copy(dst_h.at[pl.ds(ebase + k * _BD, _BD)], idx_v)
        pltpu.sync_copy(ones_v, acc.at[idx_v], add=True)
        return _

    lax.fori_loop(0, (_E // 32) // _BD, eblk, None)
    plsc.subcore_barrier()

    def wout(out_h):
        def wblk(k, _):
            pltpu.sync_copy(acc.at[pl.ds(base_r + k * _BD, _BD)], buf_v)
            pltpu.sync_copy(buf_v, out_h.at[pl.ds(base_r + k * _BD, _BD)])
            return _

        lax.fori_loop(0, _NBD, wblk, None)
        pltpu.sync_copy(acc.at[pl.ds(base_r + _NBD * _BD, _TAILD)], buf_t)
        pltpu.sync_copy(buf_t, out_h.at[pl.ds(base_r + _NBD * _BD, _TAILD)])

    pl.when(c == 0)(lambda: wout(d0_h))
    pl.when(c == 1)(lambda: wout(d1_h))


def _make_deg_kernel():
    return pl.kernel(
        _deg_body,
        out_type=(
            jax.ShapeDtypeStruct((_NP,), jnp.float32),
            jax.ShapeDtypeStruct((_NP,), jnp.float32),
        ),
        mesh=_mesh,
        compiler_params=pltpu.CompilerParams(use_tc_tiling_on_sc=False),
        scratch_types=[
            pltpu.VMEM_SHARED((_NP,), jnp.float32),
            pltpu.VMEM((_BD,), jnp.int32),
            pltpu.VMEM((_BD,), jnp.float32),
            pltpu.VMEM((_BD,), jnp.float32),
            pltpu.VMEM((_TAILD,), jnp.float32),
        ],
    )


def _init_acc2d(src_hbm, acc, rows_v, rows_t, base_r, fixed_block):
    def blk(k, _):
        r0 = 0 if fixed_block else base_r + k * _B
        pltpu.sync_copy(src_hbm.at[pl.ds(r0, _B), :], rows_v)
        pltpu.sync_copy(rows_v, acc.at[pl.ds(base_r + k * _B, _B), :])
        return _

    lax.fori_loop(0, _NBI, blk, None)
    r0 = 0 if fixed_block else base_r + _NBI * _B
    pltpu.sync_copy(src_hbm.at[pl.ds(r0, _TAIL), :], rows_t)
    pltpu.sync_copy(rows_t, acc.at[pl.ds(base_r + _NBI * _B, _TAIL), :])


def _writeout_acc2d(acc, out_hbm, rows_v, rows_t, base_r):
    def blk(k, _):
        pltpu.sync_copy(acc.at[pl.ds(base_r + k * _B, _B), :], rows_v)
        pltpu.sync_copy(rows_v, out_hbm.at[pl.ds(base_r + k * _B, _B), :])
        return _

    lax.fori_loop(0, _NBI, blk, None)
    pltpu.sync_copy(acc.at[pl.ds(base_r + _NBI * _B, _TAIL), :], rows_t)
    pltpu.sync_copy(rows_t, out_hbm.at[pl.ds(base_r + _NBI * _B, _TAIL), :])


def _edge_loop(tab, src_h, dst_h, acc,
               idx_s0, idx_s1, idx_d, rows0, rows1, sem0, sem1,
               ebase, nblk):
    npair = nblk // 2
    leftover = nblk % 2

    pltpu.sync_copy(src_h.at[pl.ds(ebase, _B)], idx_s0)
    pltpu.async_copy(tab.at[idx_s0], rows0, sem0)

    def pair(j, _):
        e0 = ebase + (2 * j) * _B
        e1 = e0 + _B
        pltpu.sync_copy(src_h.at[pl.ds(e1, _B)], idx_s1)
        pltpu.async_copy(tab.at[idx_s1], rows1, sem1)
        pltpu.make_async_copy(tab.at[idx_s0], rows0, sem0).wait()
        pltpu.sync_copy(dst_h.at[pl.ds(e0, _B)], idx_d)
        pltpu.sync_copy(rows0, acc.at[idx_d], add=True)

        @pl.when(j < npair - 1)
        def _pref():
            pltpu.sync_copy(src_h.at[pl.ds(e1 + _B, _B)], idx_s0)
            pltpu.async_copy(tab.at[idx_s0], rows0, sem0)

        pltpu.make_async_copy(tab.at[idx_s1], rows1, sem1).wait()
        pltpu.sync_copy(dst_h.at[pl.ds(e1, _B)], idx_d)
        pltpu.sync_copy(rows1, acc.at[idx_d], add=True)
        return _

    lax.fori_loop(0, npair, pair, None)

    if leftover:
        e0 = ebase + (nblk - 1) * _B
        pltpu.sync_copy(src_h.at[pl.ds(e0, _B)], idx_s0)
        pltpu.async_copy(tab.at[idx_s0], rows0, sem0).wait()
        pltpu.sync_copy(dst_h.at[pl.ds(e0, _B)], idx_d)
        pltpu.sync_copy(rows0, acc.at[idx_d], add=True)


def _prop1_body(t0_h, t1_h, t2_h, t3_h, src_h, dst_h,
                o0_h, o1_h, o2_h, o3_h,
                acc, idx_s0, idx_s1, idx_d, rows0, rows1, rows_t,
                sem0, sem1):
    c = lax.axis_index("c")
    s = lax.axis_index("s")
    base_r = s * _SLICE
    nblk = _E // 16 // _B
    ebase = s * (_E // 16)

    def run(tab, out_h):
        _init_acc2d(tab, acc, rows0, rows_t, base_r, False)
        plsc.subcore_barrier()
        _edge_loop(tab, src_h, dst_h, acc,
                   idx_s0, idx_s1, idx_d, rows0, rows1, sem0, sem1,
                   ebase, nblk)
        plsc.subcore_barrier()
        _writeout_acc2d(acc, out_h, rows0, rows_t, base_r)
        plsc.subcore_barrier()

    def core0():
        run(t0_h, o0_h)
        run(t1_h, o1_h)

    def core1():
        run(t2_h, o2_h)
        run(t3_h, o3_h)

    pl.when(c == 0)(core0)
    pl.when(c == 1)(core1)


def _make_prop1():
    return pl.kernel(
        _prop1_body,
        out_type=tuple(
            jax.ShapeDtypeStruct((_NP, 16), jnp.float32) for _ in range(4)
        ),
        mesh=_mesh,
        compiler_params=pltpu.CompilerParams(use_tc_tiling_on_sc=False),
        scratch_types=[
            pltpu.VMEM_SHARED((_NP, 16), jnp.float32),
            pltpu.VMEM((_B,), jnp.int32),
            pltpu.VMEM((_B,), jnp.int32),
            pltpu.VMEM((_B,), jnp.int32),
            pltpu.VMEM((_B, 16), jnp.float32),
            pltpu.VMEM((_B, 16), jnp.float32),
            pltpu.VMEM((_TAIL, 16), jnp.float32),
            pltpu.SemaphoreType.DMA,
            pltpu.SemaphoreType.DMA,
        ],
    )


def _prop2_body(tab_h, zeros_h, src_h, dst_h, p0_h, p1_h,
                acc, idx_s0, idx_s1, idx_d, rows0, rows1, rows_t,
                sem0, sem1):
    c = lax.axis_index("c")
    s = lax.axis_index("s")
    base_r = s * _SLICE
    epc = _E // 2
    nblk = epc // 16 // _B

    def run(init_h, out_h, fixed, ebase0):
        _init_acc2d(init_h, acc, rows0, rows_t, base_r, fixed)
        plsc.subcore_barrier()
        _edge_loop(tab_h, src_h, dst_h, acc,
                   idx_s0, idx_s1, idx_d, rows0, rows1, sem0, sem1,
                   ebase0 + s * (epc // 16), nblk)
        plsc.subcore_barrier()
        _writeout_acc2d(acc, out_h, rows0, rows_t, base_r)

    pl.when(c == 0)(lambda: run(tab_h, p0_h, False, 0))
    pl.when(c == 1)(lambda: run(zeros_h, p1_h, True, epc))


def _make_prop2():
    return pl.kernel(
        _prop2_body,
        out_type=(
            jax.ShapeDtypeStruct((_NP, 16), jnp.float32),
            jax.ShapeDtypeStruct((_NP, 16), jnp.float32),
        ),
        mesh=_mesh,
        compiler_params=pltpu.CompilerParams(use_tc_tiling_on_sc=False),
        scratch_types=[
            pltpu.VMEM_SHARED((_NP, 16), jnp.float32),
            pltpu.VMEM((_B,), jnp.int32),
            pltpu.VMEM((_B,), jnp.int32),
            pltpu.VMEM((_B,), jnp.int32),
            pltpu.VMEM((_B, 16), jnp.float32),
            pltpu.VMEM((_B, 16), jnp.float32),
            pltpu.VMEM((_TAIL, 16), jnp.float32),
            pltpu.SemaphoreType.DMA,
            pltpu.SemaphoreType.DMA,
        ],
    )


def _mm_kernel(x_ref, w_ref, o_ref):
    o_ref[...] = jnp.dot(x_ref[...], w_ref[...],
                         preferred_element_type=jnp.float32)


def _scale_split_kernel(h_ref, d0_ref, d1_ref, o0, o1, o2, o3):
    dinv = lax.rsqrt(d0_ref[...] + d1_ref[...] + 1.0)
    g = h_ref[...] * dinv
    o0[...] = g[:, 0:16]
    o1[...] = g[:, 16:32]
    o2[...] = g[:, 32:48]
    o3[...] = g[:, 48:64]


def _mid_kernel(a0, a1, a2, a3, d0_ref, d1_ref, b1_ref, w2_ref, o_ref):
    agg = jnp.concatenate([a0[...], a1[...], a2[...], a3[...]], axis=1)
    dinv = lax.rsqrt(d0_ref[...] + d1_ref[...] + 1.0)
    h1 = jax.nn.relu(agg * dinv + b1_ref[...])
    g2 = jnp.dot(h1, w2_ref[...], preferred_element_type=jnp.float32)
    o_ref[...] = g2 * dinv


def _final_kernel(p0, p1, d0_ref, d1_ref, b2_ref, o_ref):
    dinv = lax.rsqrt(d0_ref[...] + d1_ref[...] + 1.0)
    v = (p0[...] + p1[...]) * dinv
    o_ref[...] = jax.nn.sigmoid(v[:, 0:2] + b2_ref[...])


def kernel(x, edge_index, W1, b1, W2, b2):
    src = edge_index[0]
    dst = edge_index[1]

    ones1 = jnp.ones((_BD,), jnp.float32)
    zeros1 = jnp.zeros((_BD,), jnp.float32)
    zeros2 = jnp.zeros((_B, 16), jnp.float32)

    grid = _N // 2000
    tab_sds = jax.ShapeDtypeStruct((_NP, 16), jnp.float32)
    tab_spec = pl.BlockSpec((2000, 16), lambda i: (i, 0))
    dinv_spec = pl.BlockSpec((2000, 1), lambda i: (i, 0))

    W1p = jnp.pad(W1, ((0, 0), (0, 14)))
    h = pl.pallas_call(
        _mm_kernel,
        grid=(grid,),
        in_specs=[
            pl.BlockSpec((2000, 768), lambda i: (i, 0)),
            pl.BlockSpec((768, 64), lambda i: (0, 0)),
        ],
        out_specs=pl.BlockSpec((2000, 64), lambda i: (i, 0)),
        out_shape=jax.ShapeDtypeStruct((_N, 64), jnp.float32),
    )(x, W1p)

    d0, d1 = _make_deg_kernel()(dst, ones1, zeros1)
    d0r = d0.reshape(_NP, 1)
    d1r = d1.reshape(_NP, 1)

    g1c = pl.pallas_call(
        _scale_split_kernel,
        grid=(grid,),
        in_specs=[
            pl.BlockSpec((2000, 64), lambda i: (i, 0)),
            dinv_spec,
            dinv_spec,
        ],
        out_specs=[tab_spec] * 4,
        out_shape=[tab_sds] * 4,
    )(h, d0r, d1r)

    agg = _make_prop1()(g1c[0], g1c[1], g1c[2], g1c[3], src, dst)

    b1p = jnp.pad(b1, (0, 14)).reshape(1, 64)
    W2p = jnp.pad(W2, ((0, 14), (0, 14)))
    g2 = pl.pallas_call(
        _mid_kernel,
        grid=(grid,),
        in_specs=[tab_spec, tab_spec, tab_spec, tab_spec,
                  dinv_spec, dinv_spec,
                  pl.BlockSpec((1, 64), lambda i: (0, 0)),
                  pl.BlockSpec((64, 16), lambda i: (0, 0))],
        out_specs=tab_spec,
        out_shape=tab_sds,
    )(agg[0], agg[1], agg[2], agg[3], d0r, d1r, b1p, W2p)

    p0, p1 = _make_prop2()(g2, zeros2, src, dst)

    b2r = b2.reshape(1, 2)
    out = pl.pallas_call(
        _final_kernel,
        grid=(grid,),
        in_specs=[tab_spec, tab_spec, dinv_spec, dinv_spec,
                  pl.BlockSpec((1, 2), lambda i: (0, 0))],
        out_specs=pl.BlockSpec((2000, 2), lambda i: (i, 0)),
        out_shape=jax.ShapeDtypeStruct((_N, 2), jnp.float32),
    )(p0, p1, d0r, d1r, b2r)
    return out


# speedup vs baseline: 1.6016x; 1.0983x over previous
import functools

import jax
import jax.numpy as jnp
from jax import lax
from jax.experimental import pallas as pl
from jax.experimental.pallas import tpu as pltpu
from jax.experimental.pallas import tpu_sc as plsc

_N = 100000
_E = 1600000
_NP = 100096
_SLICE = _NP // 16
_B = 1000
_NBI = _SLICE // _B
_TAIL = _SLICE - _NBI * _B
_BD = 2000
_NBD = _SLICE // _BD
_TAILD = _SLICE - _NBD * _BD

_mesh = plsc.VectorSubcoreMesh(core_axis_name="c", subcore_axis_name="s")


def _init_acc2d(src_hbm, acc, rows_v, rows_t, base_r, fixed_block):
    def blk(k, _):
        r0 = 0 if fixed_block else base_r + k * _B
        pltpu.sync_copy(src_hbm.at[pl.ds(r0, _B), :], rows_v)
        pltpu.sync_copy(rows_v, acc.at[pl.ds(base_r + k * _B, _B), :])
        return _

    lax.fori_loop(0, _NBI, blk, None)
    r0 = 0 if fixed_block else base_r + _NBI * _B
    pltpu.sync_copy(src_hbm.at[pl.ds(r0, _TAIL), :], rows_t)
    pltpu.sync_copy(rows_t, acc.at[pl.ds(base_r + _NBI * _B, _TAIL), :])


def _writeout_acc2d(acc, out_hbm, rows_v, rows_t, base_r):
    def blk(k, _):
        pltpu.sync_copy(acc.at[pl.ds(base_r + k * _B, _B), :], rows_v)
        pltpu.sync_copy(rows_v, out_hbm.at[pl.ds(base_r + k * _B, _B), :])
        return _

    lax.fori_loop(0, _NBI, blk, None)
    pltpu.sync_copy(acc.at[pl.ds(base_r + _NBI * _B, _TAIL), :], rows_t)
    pltpu.sync_copy(rows_t, out_hbm.at[pl.ds(base_r + _NBI * _B, _TAIL), :])


def _deg_body(dst_h, ones_h, zeros_h, d0_h, d1_h,
              acc, idx_v, ones_v, buf_v, buf_t):
    c = lax.axis_index("c")
    s = lax.axis_index("s")
    base_r = s * _SLICE

    pltpu.sync_copy(ones_h, ones_v)

    def iblk(k, _):
        pltpu.sync_copy(zeros_h.at[pl.ds(0, _BD)], buf_v)
        pltpu.sync_copy(buf_v, acc.at[pl.ds(base_r + k * _BD, _BD)])
        return _

    lax.fori_loop(0, _NBD, iblk, None)
    pltpu.sync_copy(zeros_h.at[pl.ds(0, _TAILD)], buf_t)
    pltpu.sync_copy(buf_t, acc.at[pl.ds(base_r + _NBD * _BD, _TAILD)])
    plsc.subcore_barrier()

    ebase = (c * 16 + s) * (_E // 32)

    def eblk(k, _):
        pltpu.sync_copy(dst_h.at[pl.ds(ebase + k * _BD, _BD)], idx_v)
        pltpu.sync_copy(ones_v, acc.at[idx_v], add=True)
        return _

    lax.fori_loop(0, (_E // 32) // _BD, eblk, None)
    plsc.subcore_barrier()

    def wout(out_h):
        def wblk(k, _):
            pltpu.sync_copy(acc.at[pl.ds(base_r + k * _BD, _BD)], buf_v)
            pltpu.sync_copy(buf_v, out_h.at[pl.ds(base_r + k * _BD, _BD)])
            return _

        lax.fori_loop(0, _NBD, wblk, None)
        pltpu.sync_copy(acc.at[pl.ds(base_r + _NBD * _BD, _TAILD)], buf_t)
        pltpu.sync_copy(buf_t, out_h.at[pl.ds(base_r + _NBD * _BD, _TAILD)])

    pl.when(c == 0)(lambda: wout(d0_h))
    pl.when(c == 1)(lambda: wout(d1_h))


def _make_deg_kernel():
    return pl.kernel(
        _deg_body,
        out_type=(
            jax.ShapeDtypeStruct((_NP,), jnp.float32),
            jax.ShapeDtypeStruct((_NP,), jnp.float32),
        ),
        mesh=_mesh,
        compiler_params=pltpu.CompilerParams(use_tc_tiling_on_sc=False),
        scratch_types=[
            pltpu.VMEM_SHARED((_NP,), jnp.float32),
            pltpu.VMEM((_BD,), jnp.int32),
            pltpu.VMEM((_BD,), jnp.float32),
            pltpu.VMEM((_BD,), jnp.float32),
            pltpu.VMEM((_TAILD,), jnp.float32),
        ],
    )


def _prop_body(tab0_h, tab1_h, init1_h, src_h, dst_h, out0_h, out1_h,
               acc, idx_s, idx_d, rows_v, rows_t, sem,
               *, edges_per_core, init1_fixed):
    c = lax.axis_index("c")
    s = lax.axis_index("s")
    base_r = s * _SLICE
    nblk = edges_per_core // 16 // _B

    def run(tab, init_h, out_h, fixed, ebase0):
        _init_acc2d(init_h, acc, rows_v, rows_t, base_r, fixed)
        plsc.subcore_barrier()

        ebase = ebase0 + s * (edges_per_core // 16)

        def eblk(k, _):
            off = ebase + k * _B
            pltpu.sync_copy(src_h.at[pl.ds(off, _B)], idx_s)
            pltpu.async_copy(tab.at[idx_s], rows_v, sem).wait()
            pltpu.sync_copy(dst_h.at[pl.ds(off, _B)], idx_d)
            pltpu.sync_copy(rows_v, acc.at[idx_d], add=True)
            return _

        lax.fori_loop(0, nblk, eblk, None)
        plsc.subcore_barrier()
        _writeout_acc2d(acc, out_h, rows_v, rows_t, base_r)

    if init1_fixed:
        pl.when(c == 0)(lambda: run(tab0_h, tab0_h, out0_h, False, 0))
        pl.when(c == 1)(lambda: run(tab1_h, init1_h, out1_h, True,
                                    edges_per_core))
    else:
        pl.when(c == 0)(lambda: run(tab0_h, tab0_h, out0_h, False, 0))
        pl.when(c == 1)(lambda: run(tab1_h, tab1_h, out1_h, False, 0))


def _make_prop(edges_per_core, init1_fixed):
    return pl.kernel(
        functools.partial(_prop_body, edges_per_core=edges_per_core,
                          init1_fixed=init1_fixed),
        out_type=(
            jax.ShapeDtypeStruct((_NP, 16), jnp.float32),
            jax.ShapeDtypeStruct((_NP, 16), jnp.float32),
        ),
        mesh=_mesh,
        compiler_params=pltpu.CompilerParams(use_tc_tiling_on_sc=False),
        scratch_types=[
            pltpu.VMEM_SHARED((_NP, 16), jnp.float32),
            pltpu.VMEM((_B,), jnp.int32),
            pltpu.VMEM((_B,), jnp.int32),
            pltpu.VMEM((_B, 16), jnp.float32),
            pltpu.VMEM((_TAIL, 16), jnp.float32),
            pltpu.SemaphoreType.DMA,
        ],
    )


def _dinv_kernel(d0_ref, d1_ref, o_ref):
    o_ref[...] = lax.rsqrt(d0_ref[...] + d1_ref[...] + 1.0)


def _scale_split_kernel(x_ref, w_ref, dinv_ref, o0, o1, o2, o3):
    h = jnp.dot(x_ref[...], w_ref[...], preferred_element_type=jnp.float32)
    g = h * dinv_ref[...]
    o0[...] = g[:, 0:16]
    o1[...] = g[:, 16:32]
    o2[...] = g[:, 32:48]
    o3[...] = g[:, 48:64]


def _mid_kernel(a0, a1, a2, a3, dinv_ref, b1_ref, w2_ref, o_ref):
    agg = jnp.concatenate([a0[...], a1[...], a2[...], a3[...]], axis=1)
    h1 = jax.nn.relu(agg * dinv_ref[...] + b1_ref[...])
    g2 = jnp.dot(h1, w2_ref[...], preferred_element_type=jnp.float32)
    o_ref[...] = g2 * dinv_ref[...]


def _final_kernel(p0, p1, dinv_ref, b2_ref, o_ref):
    v = (p0[...] + p1[...]) * dinv_ref[...]
    o_ref[...] = jax.nn.sigmoid(v[:, 0:2] + b2_ref[...])


def kernel(x, edge_index, W1, b1, W2, b2):
    src = edge_index[0]
    dst = edge_index[1]

    ones1 = jnp.ones((_BD,), jnp.float32)
    zeros1 = jnp.zeros((_BD,), jnp.float32)
    zeros2 = jnp.zeros((_B, 16), jnp.float32)

    d0, d1 = _make_deg_kernel()(dst, ones1, zeros1)

    dinvp = pl.pallas_call(
        _dinv_kernel,
        out_shape=jax.ShapeDtypeStruct((782, 128), jnp.float32),
    )(d0.reshape(782, 128), d1.reshape(782, 128))
    dinv2d = dinvp.reshape(_NP, 1)

    W1p = jnp.pad(W1, ((0, 0), (0, 14)))
    grid = _N // 2000
    tab_sds = jax.ShapeDtypeStruct((_NP, 16), jnp.float32)
    tab_spec = pl.BlockSpec((2000, 16), lambda i: (i, 0))
    g1c = pl.pallas_call(
        _scale_split_kernel,
        grid=(grid,),
        in_specs=[
            pl.BlockSpec((2000, 768), lambda i: (i, 0)),
            pl.BlockSpec((768, 64), lambda i: (0, 0)),
            pl.BlockSpec((2000, 1), lambda i: (i, 0)),
        ],
        out_specs=[tab_spec] * 4,
        out_shape=[tab_sds] * 4,
    )(x, W1p, dinv2d)

    prop_chunk = _make_prop(_E, init1_fixed=False)
    agg0, agg1 = prop_chunk(g1c[0], g1c[1], zeros2, src, dst)
    agg2, agg3 = prop_chunk(g1c[2], g1c[3], zeros2, src, dst)

    b1p = jnp.pad(b1, (0, 14)).reshape(1, 64)
    W2p = jnp.pad(W2, ((0, 14), (0, 14)))
    g2 = pl.pallas_call(
        _mid_kernel,
        grid=(grid,),
        in_specs=[tab_spec, tab_spec, tab_spec, tab_spec,
                  pl.BlockSpec((2000, 1), lambda i: (i, 0)),
                  pl.BlockSpec((1, 64), lambda i: (0, 0)),
                  pl.BlockSpec((64, 16), lambda i: (0, 0))],
        out_specs=tab_spec,
        out_shape=tab_sds,
    )(agg0, agg1, agg2, agg3, dinv2d, b1p, W2p)

    prop_half = _make_prop(_E // 2, init1_fixed=True)
    p0, p1 = prop_half(g2, g2, zeros2, src, dst)

    b2r = b2.reshape(1, 2)
    out = pl.pallas_call(
        _final_kernel,
        grid=(grid,),
        in_specs=[tab_spec, tab_spec,
                  pl.BlockSpec((2000, 1), lambda i: (i, 0)),
                  pl.BlockSpec((1, 2), lambda i: (0, 0))],
        out_specs=pl.BlockSpec((2000, 2), lambda i: (i, 0)),
        out_shape=jax.ShapeDtypeStruct((_N, 2), jnp.float32),
    )(p0, p1, dinv2d, b2r)
    return out
